# baseline (speedup 1.0000x reference)
import functools

import jax
import jax.numpy as jnp
from jax.experimental import pallas as pl
from jax.experimental.pallas import tpu as pltpu

_SUPPORT_CHUNKS = 10
_RING = 3
_OUT_SLOTS = 2
_UNROLL = 6
_BM = 400
_SMALL = 120
_N_EDGE = 5


def _schedule(m):
    steps = []
    off = 0
    for _ in range(_N_EDGE):
        steps.append((off, _SMALL))
        off += _SMALL
    end_start = m - _N_EDGE * _SMALL
    while off < end_start:
        steps.append((off, _BM))
        off += _BM
    for _ in range(_N_EDGE):
        steps.append((off, _SMALL))
        off += _SMALL
    assert off == m
    return steps


def _gcn_body(adj_hbm, x_hbm, w_ref, out_hbm,
              st_ref, wb_ref, support_ref, ring_ref, outv_ref,
              sem_x0, sem_x1, sem_r0, sem_r1, sem_r2,
              sem_o0, sem_o1):
    m, k = adj_hbm.shape
    ch = x_hbm.shape[0] // _SUPPORT_CHUNKS
    xsems = (sem_x0, sem_x1)
    rsems = (sem_r0, sem_r1, sem_r2)
    osems = (sem_o0, sem_o1)
    steps = _schedule(m)
    nb = len(steps)

    def adj_copy(off, rows, rs):
        return pltpu.make_async_copy(
            adj_hbm.at[pl.ds(off, rows), :],
            ring_ref.at[rs, pl.ds(0, rows), :], rsems[rs])

    def out_copy(off, rows, os):
        return pltpu.make_async_copy(
            outv_ref.at[os, pl.ds(0, rows), :],
            out_hbm.at[pl.ds(off, rows), :], osems[os])

    wb_ref[...] = w_ref[...].astype(jnp.bfloat16)
    for s in range(2):
        pltpu.make_async_copy(
            x_hbm.at[pl.ds(s * ch, ch), :], st_ref.at[s], xsems[s]).start()
    for c in range(_SUPPORT_CHUNKS):
        s = c % 2
        pltpu.make_async_copy(
            x_hbm.at[pl.ds(c * ch, ch), :], st_ref.at[s], xsems[s]).wait()
        support_ref[pl.ds(c * ch, ch), :] = jnp.dot(
            st_ref[s].astype(jnp.bfloat16),
            wb_ref[...],
            preferred_element_type=jnp.float32,
        ).astype(jnp.bfloat16)
        if c + 2 < _SUPPORT_CHUNKS:
            pltpu.make_async_copy(
                x_hbm.at[pl.ds((c + 2) * ch, ch), :], st_ref.at[s],
                xsems[s]).start()

    for idx in range(_RING):
        off, rows = steps[idx]
        adj_copy(off, rows, idx % _RING).start()

    def static_step(idx):
        off, rows = steps[idx]
        rs, os = idx % _RING, idx % _OUT_SLOTS
        adj_copy(off, rows, rs).wait()
        if idx >= _OUT_SLOTS:
            poff, prows = steps[idx - _OUT_SLOTS]
            out_copy(poff, prows, os).wait()
        outv_ref[os, pl.ds(0, rows), :] = jnp.dot(
            ring_ref[rs, pl.ds(0, rows), :].astype(jnp.bfloat16),
            support_ref[...],
            preferred_element_type=jnp.float32,
        )
        out_copy(off, rows, os).start()
        if idx + _RING < nb:
            noff, nrows = steps[idx + _RING]
            adj_copy(noff, nrows, rs).start()

    mid_lo = _N_EDGE
    mid_hi = nb - _N_EDGE
    lo_min = mid_lo + _OUT_SLOTS
    loop_lo = ((lo_min + _UNROLL - 1) // _UNROLL) * _UNROLL
    loop_hi_limit = mid_hi - _RING
    if loop_hi_limit > loop_lo:
        loop_hi = loop_lo + ((loop_hi_limit - loop_lo) // _UNROLL) * _UNROLL
    else:
        loop_lo = loop_hi = min(loop_lo, nb)

    for idx in range(0, loop_lo):
        static_step(idx)

    def outer(o, carry):
        base = loop_lo + _UNROLL * o
        base_off = steps[mid_lo][0] + (base - mid_lo) * _BM
        for j in range(_UNROLL):
            idx_off = base_off + j * _BM
            rs = j % _RING
            os = j % _OUT_SLOTS
            adj_copy(idx_off, _BM, rs).wait()
            out_copy(idx_off - _OUT_SLOTS * _BM, _BM, os).wait()
            outv_ref[os] = jnp.dot(
                ring_ref[rs].astype(jnp.bfloat16),
                support_ref[...],
                preferred_element_type=jnp.float32,
            )
            out_copy(idx_off, _BM, os).start()
            adj_copy(idx_off + _RING * _BM, _BM, rs).start()
        return carry

    n_rounds = (loop_hi - loop_lo) // _UNROLL
    jax.lax.fori_loop(0, n_rounds, outer, 0)

    for idx in range(loop_hi, nb):
        static_step(idx)

    for idx in range(max(nb - _OUT_SLOTS, 0), nb):
        off, rows = steps[idx]
        out_copy(off, rows, idx % _OUT_SLOTS).wait()


@jax.jit
def _gcn(adj, x, W):
    m, k = adj.shape
    d_in, d_out = W.shape
    return pl.pallas_call(
        _gcn_body,
        in_specs=[
            pl.BlockSpec(memory_space=pl.ANY),
            pl.BlockSpec(memory_space=pl.ANY),
            pl.BlockSpec((d_in, d_out), lambda: (0, 0)),
        ],
        out_specs=pl.BlockSpec(memory_space=pl.ANY),
        out_shape=jax.ShapeDtypeStruct((m, d_out), jnp.float32),
        scratch_shapes=[
            pltpu.VMEM((2, x.shape[0] // _SUPPORT_CHUNKS, d_in), jnp.float32),
            pltpu.VMEM((d_in, d_out), jnp.bfloat16),
            pltpu.VMEM((x.shape[0], d_out), jnp.bfloat16),
            pltpu.VMEM((_RING, _BM, k), jnp.float32),
            pltpu.VMEM((_OUT_SLOTS, _BM, d_out), jnp.float32),
            pltpu.SemaphoreType.DMA,
            pltpu.SemaphoreType.DMA,
            pltpu.SemaphoreType.DMA,
            pltpu.SemaphoreType.DMA,
            pltpu.SemaphoreType.DMA,
            pltpu.SemaphoreType.DMA,
            pltpu.SemaphoreType.DMA,
        ],
        compiler_params=pltpu.CompilerParams(
            vmem_limit_bytes=66_900_000,
        ),
    )(adj, x, W)


def kernel(adj, x, W):
    return _gcn(adj, x, W)


# speedup vs baseline: 1.0622x; 1.0622x over previous
import functools

import jax
import jax.numpy as jnp
from jax.experimental import pallas as pl
from jax.experimental.pallas import tpu as pltpu

_SUPPORT_CHUNKS = 10
_RING = 5
_OUT_SLOTS = 2
_UNROLL = 10


def _gcn_body(adj_hbm, x_hbm, w_ref, out_hbm,
              st_ref, wb_ref, support_ref, ring_ref, outv_ref,
              sem_x0, sem_x1, sem_r0, sem_r1, sem_r2, sem_r3, sem_r4,
              sem_o0, sem_o1):
    m, k = adj_hbm.shape
    bm = ring_ref.shape[1]
    nb = m // bm
    ch = x_hbm.shape[0] // _SUPPORT_CHUNKS
    xsems = (sem_x0, sem_x1)
    rsems = (sem_r0, sem_r1, sem_r2, sem_r3, sem_r4)
    osems = (sem_o0, sem_o1)

    wb_ref[...] = w_ref[...].astype(jnp.bfloat16)
    for s in range(2):
        pltpu.make_async_copy(
            x_hbm.at[pl.ds(s * ch, ch), :], st_ref.at[s], xsems[s]).start()
    for c in range(_SUPPORT_CHUNKS):
        s = c % 2
        pltpu.make_async_copy(
            x_hbm.at[pl.ds(c * ch, ch), :], st_ref.at[s], xsems[s]).wait()
        support_ref[pl.ds(c * ch, ch), :] = jnp.dot(
            st_ref[s].astype(jnp.bfloat16),
            wb_ref[...],
            preferred_element_type=jnp.float32,
        ).astype(jnp.bfloat16)
        if c + 2 < _SUPPORT_CHUNKS:
            pltpu.make_async_copy(
                x_hbm.at[pl.ds((c + 2) * ch, ch), :], st_ref.at[s],
                xsems[s]).start()

    for s in range(_RING):
        pltpu.make_async_copy(
            adj_hbm.at[pl.ds(s * bm, bm), :], ring_ref.at[s],
            rsems[s]).start()

    def step(i, rs, os):
        pltpu.make_async_copy(
            adj_hbm.at[pl.ds(i * bm, bm), :], ring_ref.at[rs],
            rsems[rs]).wait()

        @pl.when(i >= _OUT_SLOTS)
        def _():
            pltpu.make_async_copy(
                outv_ref.at[os],
                out_hbm.at[pl.ds((i - _OUT_SLOTS) * bm, bm), :],
                osems[os]).wait()

        outv_ref[os] = jnp.dot(
            ring_ref[rs].astype(jnp.bfloat16),
            support_ref[...],
            preferred_element_type=jnp.float32,
        )
        pltpu.make_async_copy(
            outv_ref.at[os], out_hbm.at[pl.ds(i * bm, bm), :],
            osems[os]).start()

        @pl.when(i + _RING < nb)
        def _():
            pltpu.make_async_copy(
                adj_hbm.at[pl.ds((i + _RING) * bm, bm), :], ring_ref.at[rs],
                rsems[rs]).start()

    def outer(o, carry):
        for j in range(_UNROLL):
            step(_UNROLL * o + j, j % _RING, j % _OUT_SLOTS)
        return carry

    n_full = nb // _UNROLL
    jax.lax.fori_loop(0, n_full, outer, 0)
    for i in range(n_full * _UNROLL, nb):
        step(i, i % _RING, i % _OUT_SLOTS)

    for i in range(max(nb - _OUT_SLOTS, 0), nb):
        pltpu.make_async_copy(
            outv_ref.at[i % _OUT_SLOTS], out_hbm.at[pl.ds(i * bm, bm), :],
            osems[i % _OUT_SLOTS]).wait()


@functools.partial(jax.jit, static_argnames=("block_m",))
def _gcn(adj, x, W, block_m=200):
    m, k = adj.shape
    d_in, d_out = W.shape
    bm = min(block_m, m)
    return pl.pallas_call(
        _gcn_body,
        in_specs=[
            pl.BlockSpec(memory_space=pl.ANY),
            pl.BlockSpec(memory_space=pl.ANY),
            pl.BlockSpec((d_in, d_out), lambda: (0, 0)),
        ],
        out_specs=pl.BlockSpec(memory_space=pl.ANY),
        out_shape=jax.ShapeDtypeStruct((m, d_out), jnp.float32),
        scratch_shapes=[
            pltpu.VMEM((2, x.shape[0] // _SUPPORT_CHUNKS, d_in), jnp.float32),
            pltpu.VMEM((d_in, d_out), jnp.bfloat16),
            pltpu.VMEM((x.shape[0], d_out), jnp.bfloat16),
            pltpu.VMEM((_RING, bm, k), jnp.float32),
            pltpu.VMEM((_OUT_SLOTS, bm, d_out), jnp.float32),
            pltpu.SemaphoreType.DMA,
            pltpu.SemaphoreType.DMA,
            pltpu.SemaphoreType.DMA,
            pltpu.SemaphoreType.DMA,
            pltpu.SemaphoreType.DMA,
            pltpu.SemaphoreType.DMA,
            pltpu.SemaphoreType.DMA,
            pltpu.SemaphoreType.DMA,
            pltpu.SemaphoreType.DMA,
        ],
        compiler_params=pltpu.CompilerParams(
            vmem_limit_bytes=66_900_000,
        ),
    )(adj, x, W)


def kernel(adj, x, W):
    return _gcn(adj, x, W)
